# baseline (speedup 1.0000x reference)
import functools

import jax
import jax.numpy as jnp
from jax import lax
from jax.experimental import pallas as pl
from jax.experimental.pallas import tpu as pltpu
from jax.experimental.pallas import tpu_sc as plsc

_N = 10000
_D = 128
_H = 128
_C = 64
_P = 128

_NP = 10112
_NC = 2
_NS = 16
_NW = _NC * _NS
_CHUNK = 128
_ROWS_PT = _NP // _NS


def _elu(x):
    return jnp.where(x > 0, x, jnp.exp(x) - 1.0)


def _deg_body(row_hbm, out_hbm, idx0_v, idx1_v, idx2_v, idx3_v,
              ones_v, zrow_v, deg_sh,
              i0, i1, i2, i3, s0, s1, s2, s3):
    c = lax.axis_index("c")
    s = lax.axis_index("s")
    wid = c * _NS + s
    epw = row_hbm.shape[0] // _NW
    base = wid * epw
    idxs = (idx0_v, idx1_v, idx2_v, idx3_v)
    isems = (i0, i1, i2, i3)
    ssems = (s0, s1, s2, s3)

    def _z16(i, carry):
        zrow_v[pl.ds(i * 16, 16)] = jnp.zeros((16,), jnp.float32)
        return carry

    lax.fori_loop(0, 640 // 16, _z16, 0)
    for q in range(_CHUNK // 16):
        ones_v[pl.ds(q * 16, 16)] = jnp.ones((16,), jnp.float32)
    pltpu.sync_copy(zrow_v.at[pl.ds(0, _ROWS_PT)],
                    deg_sh.at[pl.ds(s * _ROWS_PT, _ROWS_PT)])
    plsc.subcore_barrier()

    def _group(g, carry):
        icps = [pltpu.async_copy(
            row_hbm.at[pl.ds(base + (g * 4 + b) * _CHUNK, _CHUNK)],
            idxs[b], isems[b]) for b in range(4)]
        scps = []
        for b in range(4):
            icps[b].wait()
            scps.append(pltpu.async_copy(
                ones_v, deg_sh.at[idxs[b]], ssems[b], add=True))
        for cp in scps:
            cp.wait()
        return carry

    lax.fori_loop(0, epw // _CHUNK // 4, _group, 0)
    plsc.subcore_barrier()

    pltpu.sync_copy(deg_sh.at[pl.ds(s * _ROWS_PT, _ROWS_PT)],
                    zrow_v.at[pl.ds(0, _ROWS_PT)])
    pltpu.sync_copy(zrow_v, out_hbm.at[c, s])


def _spmm_body(row_hbm, col_hbm, mvs_hbm, pmv_hbm,
               ridx0_v, ridx1_v, ridx2_v, ridx3_v,
               cidx0_v, cidx1_v, cidx2_v, cidx3_v,
               rows0_v, rows1_v, rows2_v, rows3_v, zbuf_v,
               acc_sh, i0, i1, i2, i3, j0, j1, j2, j3,
               g0, g1, g2, g3, s0, s1, s2, s3):
    c = lax.axis_index("c")
    s = lax.axis_index("s")
    wid = c * _NS + s
    epw = row_hbm.shape[0] // _NW
    base = wid * epw
    ridxs = (ridx0_v, ridx1_v, ridx2_v, ridx3_v)
    cidxs = (cidx0_v, cidx1_v, cidx2_v, cidx3_v)
    rows = (rows0_v, rows1_v, rows2_v, rows3_v)
    isems = (i0, i1, i2, i3)
    jsems = (j0, j1, j2, j3)
    gsems = (g0, g1, g2, g3)
    ssems = (s0, s1, s2, s3)

    def _z2d(i, carry):
        for q in range(_P // 16):
            zbuf_v[i, pl.ds(q * 16, 16)] = jnp.zeros((16,), jnp.float32)
        return carry

    lax.fori_loop(0, _CHUNK, _z2d, 0)
    for j in range(4):
        off = s * _ROWS_PT + j * _CHUNK
        pltpu.sync_copy(zbuf_v, acc_sh.at[pl.ds(off, _CHUNK)])
    off = s * _ROWS_PT + 4 * _CHUNK
    pltpu.sync_copy(zbuf_v.at[pl.ds(0, _ROWS_PT - 4 * _CHUNK)],
                    acc_sh.at[pl.ds(off, _ROWS_PT - 4 * _CHUNK)])
    plsc.subcore_barrier()

    def _group(g, carry):
        ccps = [pltpu.async_copy(
            col_hbm.at[pl.ds(base + (g * 2 + b) * _CHUNK, _CHUNK)],
            cidxs[b], isems[b]) for b in range(2)]
        rcps = [pltpu.async_copy(
            row_hbm.at[pl.ds(base + (g * 2 + b) * _CHUNK, _CHUNK)],
            ridxs[b], jsems[b]) for b in range(2)]
        scps = []
        for b in range(2):
            ccps[b].wait()
            pltpu.async_copy(mvs_hbm.at[cidxs[b]], rows[b], gsems[b]).wait()
            rcps[b].wait()
            scps.append(pltpu.async_copy(
                rows[b], acc_sh.at[ridxs[b]], ssems[b], add=True))
        for cp in scps:
            cp.wait()
        return carry

    lax.fori_loop(0, epw // _CHUNK // 2, _group, 0)
    plsc.subcore_barrier()

    for j in range(4):
        off = s * _ROWS_PT + j * _CHUNK
        pltpu.sync_copy(acc_sh.at[pl.ds(off, _CHUNK)], zbuf_v)
        pltpu.sync_copy(zbuf_v, pmv_hbm.at[c, pl.ds(off, _CHUNK)])
    off = s * _ROWS_PT + 4 * _CHUNK
    tail = _ROWS_PT - 4 * _CHUNK
    pltpu.sync_copy(acc_sh.at[pl.ds(off, tail)], zbuf_v.at[pl.ds(0, tail)])
    pltpu.sync_copy(zbuf_v.at[pl.ds(0, tail)], pmv_hbm.at[c, pl.ds(off, tail)])


def _dense_body(x_ref, deg_ref, w0m_ref, b0m_ref, w0v_ref, b0v_ref,
                w1m_ref, b1m_ref, w1v_ref, b1v_ref, mvs_ref):
    dot = functools.partial(jnp.dot, preferred_element_type=jnp.float32,
                            precision=lax.Precision.HIGHEST)
    xb = x_ref[...]
    hm = _elu(dot(xb, w0m_ref[...]) + b0m_ref[...])
    hv = jnp.maximum(dot(xb, w0v_ref[...]) + b0v_ref[...], 0.0)
    m = _elu(dot(hm, w1m_ref[...]) + b1m_ref[...])
    v = jnp.maximum(dot(hv, w1v_ref[...]) + b1v_ref[...], 0.0) + 1e-6
    att = jnp.exp(-v)
    deg = deg_ref[...]
    dinv0 = lax.rsqrt(deg)
    dinv1 = 1.0 / deg
    mvs_ref[...] = jnp.concatenate(
        [dinv0 * (m * att), dinv1 * (v * att * att)], axis=1)


def _final_body(pmv_ref, mvs_ref, deg_ref, smp_ref, out_ref):
    deg = deg_ref[...]
    dinv0 = lax.rsqrt(deg)
    dinv1 = 1.0 / deg
    tot = pmv_ref[0] + pmv_ref[1] + mvs_ref[...]
    mean = dinv0 * tot[:, :_C]
    var = dinv1 * tot[:, _C:]
    o = mean + smp_ref[...] * jnp.sqrt(var)
    o = o - jnp.max(o, axis=-1, keepdims=True)
    out_ref[...] = o - jnp.log(jnp.sum(jnp.exp(o), axis=-1, keepdims=True))


def kernel(x, adj, W0m, b0m, W0v, b0v, W1m, b1m, W1v, b1v):
    sample = jax.random.normal(jax.random.key(42), (_N, _C), dtype=jnp.float32)
    e = adj.shape[1]
    epw = -(-e // (_NW * 4 * _CHUNK)) * 4 * _CHUNK
    ep = epw * _NW

    rowp = jnp.concatenate([adj[0], jnp.full((ep - e,), _NP - 1, jnp.int32)])
    colp = jnp.concatenate([adj[1], jnp.zeros((ep - e,), jnp.int32)])

    mesh = plsc.VectorSubcoreMesh(core_axis_name="c", subcore_axis_name="s")

    deg_call = pl.kernel(
        _deg_body,
        out_type=jax.ShapeDtypeStruct((_NC, _NS, 640), jnp.float32),
        mesh=mesh,
        scratch_types=(
            [pltpu.VMEM((_CHUNK,), jnp.int32)] * 4 +
            [pltpu.VMEM((_CHUNK,), jnp.float32),
             pltpu.VMEM((640,), jnp.float32),
             pltpu.VMEM_SHARED((_NP,), jnp.float32)] +
            [pltpu.SemaphoreType.DMA] * 8
        ),
    )
    pdeg = deg_call(rowp)
    pdeg = pdeg[:, :, :_ROWS_PT].reshape(_NC, _NP)
    deg2 = (pdeg[0, :_N] + pdeg[1, :_N] + 1.0).reshape(_N, 1)

    nb = 10
    bn = _N // nb
    mvs = pl.pallas_call(
        _dense_body,
        grid=(nb,),
        in_specs=[
            pl.BlockSpec((bn, _D), lambda i: (i, 0)),
            pl.BlockSpec((bn, 1), lambda i: (i, 0)),
            pl.BlockSpec((_D, _H), lambda i: (0, 0)),
            pl.BlockSpec((1, _H), lambda i: (0, 0)),
            pl.BlockSpec((_D, _H), lambda i: (0, 0)),
            pl.BlockSpec((1, _H), lambda i: (0, 0)),
            pl.BlockSpec((_H, _C), lambda i: (0, 0)),
            pl.BlockSpec((1, _C), lambda i: (0, 0)),
            pl.BlockSpec((_H, _C), lambda i: (0, 0)),
            pl.BlockSpec((1, _C), lambda i: (0, 0)),
        ],
        out_specs=pl.BlockSpec((bn, _P), lambda i: (i, 0)),
        out_shape=jax.ShapeDtypeStruct((_N, _P), jnp.float32),
    )(x, deg2, W0m, b0m.reshape(1, _H), W0v, b0v.reshape(1, _H),
      W1m, b1m.reshape(1, _C), W1v, b1v.reshape(1, _C))

    spmm_call = pl.kernel(
        _spmm_body,
        out_type=jax.ShapeDtypeStruct((_NC, _NP, _P), jnp.float32),
        mesh=mesh,
        scratch_types=(
            [pltpu.VMEM((_CHUNK,), jnp.int32)] * 8 +
            [pltpu.VMEM((_CHUNK, _P), jnp.float32)] * 5 +
            [pltpu.VMEM_SHARED((_NP, _P), jnp.float32)] +
            [pltpu.SemaphoreType.DMA] * 16
        ),
    )
    pmv = spmm_call(rowp, colp, mvs)

    out = pl.pallas_call(
        _final_body,
        grid=(nb,),
        in_specs=[
            pl.BlockSpec((_NC, bn, _P), lambda i: (0, i, 0)),
            pl.BlockSpec((bn, _P), lambda i: (i, 0)),
            pl.BlockSpec((bn, 1), lambda i: (i, 0)),
            pl.BlockSpec((bn, _C), lambda i: (i, 0)),
        ],
        out_specs=pl.BlockSpec((bn, _C), lambda i: (i, 0)),
        out_shape=jax.ShapeDtypeStruct((_N, _C), jnp.float32),
    )(pmv, mvs, deg2, sample)
    return out


# speedup vs baseline: 1.3853x; 1.3853x over previous
import functools

import jax
import jax.numpy as jnp
from jax import lax
from jax.experimental import pallas as pl
from jax.experimental.pallas import tpu as pltpu
from jax.experimental.pallas import tpu_sc as plsc

_N = 10000
_D = 128
_H = 128
_C = 64
_P = 128

_NP = 10112
_NC = 2
_NS = 16
_NW = _NC * _NS
_CHUNK = 128
_ROWS_PT = _NP // _NS


def _elu(x):
    return jnp.where(x > 0, x, jnp.exp(x) - 1.0)


def _deg_body(row_hbm, out_hbm, idx0_v, idx1_v, idx2_v, idx3_v,
              ones_v, zrow_v, deg_sh,
              i0, i1, i2, i3, s0, s1, s2, s3):
    c = lax.axis_index("c")
    s = lax.axis_index("s")
    wid = c * _NS + s
    epw = row_hbm.shape[0] // _NW
    base = wid * epw
    idxs = (idx0_v, idx1_v, idx2_v, idx3_v)
    isems = (i0, i1, i2, i3)
    ssems = (s0, s1, s2, s3)

    def _z16(i, carry):
        zrow_v[pl.ds(i * 16, 16)] = jnp.zeros((16,), jnp.float32)
        return carry

    lax.fori_loop(0, 640 // 16, _z16, 0)
    for q in range(_CHUNK // 16):
        ones_v[pl.ds(q * 16, 16)] = jnp.ones((16,), jnp.float32)
    pltpu.sync_copy(zrow_v.at[pl.ds(0, _ROWS_PT)],
                    deg_sh.at[pl.ds(s * _ROWS_PT, _ROWS_PT)])
    plsc.subcore_barrier()

    def _group(g, carry):
        icps = [pltpu.async_copy(
            row_hbm.at[pl.ds(base + (g * 4 + b) * _CHUNK, _CHUNK)],
            idxs[b], isems[b]) for b in range(4)]
        scps = []
        for b in range(4):
            icps[b].wait()
            scps.append(pltpu.async_copy(
                ones_v, deg_sh.at[idxs[b]], ssems[b], add=True))
        for cp in scps:
            cp.wait()
        return carry

    lax.fori_loop(0, epw // _CHUNK // 4, _group, 0)
    plsc.subcore_barrier()

    pltpu.sync_copy(deg_sh.at[pl.ds(s * _ROWS_PT, _ROWS_PT)],
                    zrow_v.at[pl.ds(0, _ROWS_PT)])
    pltpu.sync_copy(zrow_v, out_hbm.at[c, s])


def _spmm_body(row_hbm, col_hbm, mvsa_hbm, mvsb_hbm, pmv_hbm,
               ridx0_v, ridx1_v, ridx2_v, ridx3_v,
               cidx0_v, cidx1_v, cidx2_v, cidx3_v,
               rows0_v, rows1_v, rows2_v, rows3_v, zbuf_v,
               acc_sh, i0, i1, i2, i3, j0, j1, j2, j3,
               g0, g1, g2, g3, s0, s1, s2, s3):
    c = lax.axis_index("c")
    s = lax.axis_index("s")
    wid = c * _NS + s
    epw = row_hbm.shape[0] // _NW
    base = wid * epw
    ridxs = (ridx0_v, ridx1_v, ridx2_v, ridx3_v)
    cidxs = (cidx0_v, cidx1_v, cidx2_v, cidx3_v)
    rows = (rows0_v, rows1_v, rows2_v, rows3_v)
    isems = (i0, i1, i2, i3)
    jsems = (j0, j1, j2, j3)
    gsems = (g0, g1, g2, g3)
    ssems = (s0, s1, s2, s3)

    def _z2d(i, carry):
        for q in range(_P // 16):
            zbuf_v[i, pl.ds(q * 16, 16)] = jnp.zeros((16,), jnp.float32)
        return carry

    lax.fori_loop(0, _CHUNK, _z2d, 0)
    for j in range(4):
        off = s * _ROWS_PT + j * _CHUNK
        pltpu.sync_copy(zbuf_v, acc_sh.at[pl.ds(off, _CHUNK)])
    off = s * _ROWS_PT + 4 * _CHUNK
    pltpu.sync_copy(zbuf_v.at[pl.ds(0, _ROWS_PT - 4 * _CHUNK)],
                    acc_sh.at[pl.ds(off, _ROWS_PT - 4 * _CHUNK)])
    plsc.subcore_barrier()

    tabs = (mvsa_hbm, mvsb_hbm)

    def _group(g, carry):
        ccps = [pltpu.async_copy(
            col_hbm.at[pl.ds(base + (g * 2 + b) * _CHUNK, _CHUNK)],
            cidxs[b], isems[b]) for b in range(2)]
        rcps = [pltpu.async_copy(
            row_hbm.at[pl.ds(base + (g * 2 + b) * _CHUNK, _CHUNK)],
            ridxs[b], jsems[b]) for b in range(2)]
        gcps = []
        for b in range(2):
            ccps[b].wait()
            gcps.append(pltpu.async_copy(
                tabs[b].at[cidxs[b]], rows[b], gsems[b]))
        for b in range(2):
            gcps[b].wait()
            rcps[b].wait()
            pltpu.async_copy(rows[b], acc_sh.at[ridxs[b]], ssems[b],
                             add=True).wait()
        return carry

    lax.fori_loop(0, epw // _CHUNK // 2, _group, 0)
    plsc.subcore_barrier()

    for j in range(4):
        off = s * _ROWS_PT + j * _CHUNK
        pltpu.sync_copy(acc_sh.at[pl.ds(off, _CHUNK)], zbuf_v)
        pltpu.sync_copy(zbuf_v, pmv_hbm.at[c, pl.ds(off, _CHUNK)])
    off = s * _ROWS_PT + 4 * _CHUNK
    tail = _ROWS_PT - 4 * _CHUNK
    pltpu.sync_copy(acc_sh.at[pl.ds(off, tail)], zbuf_v.at[pl.ds(0, tail)])
    pltpu.sync_copy(zbuf_v.at[pl.ds(0, tail)], pmv_hbm.at[c, pl.ds(off, tail)])


def _dense_body(x_ref, deg_ref, w0m_ref, b0m_ref, w0v_ref, b0v_ref,
                w1m_ref, b1m_ref, w1v_ref, b1v_ref, mvs_ref):
    dot = functools.partial(jnp.dot, preferred_element_type=jnp.float32,
                            precision=lax.Precision.HIGHEST)
    xb = x_ref[...]
    hm = _elu(dot(xb, w0m_ref[...]) + b0m_ref[...])
    hv = jnp.maximum(dot(xb, w0v_ref[...]) + b0v_ref[...], 0.0)
    m = _elu(dot(hm, w1m_ref[...]) + b1m_ref[...])
    v = jnp.maximum(dot(hv, w1v_ref[...]) + b1v_ref[...], 0.0) + 1e-6
    att = jnp.exp(-v)
    deg = deg_ref[...]
    dinv0 = lax.rsqrt(deg)
    dinv1 = 1.0 / deg
    mvs_ref[...] = jnp.concatenate(
        [dinv0 * (m * att), dinv1 * (v * att * att)], axis=1)


def _final_body(pmv_ref, mvs_ref, deg_ref, smp_ref, out_ref):
    deg = deg_ref[...]
    dinv0 = lax.rsqrt(deg)
    dinv1 = 1.0 / deg
    tot = pmv_ref[0] + pmv_ref[1] + mvs_ref[...]
    mean = dinv0 * tot[:, :_C]
    var = dinv1 * tot[:, _C:]
    o = mean + smp_ref[...] * jnp.sqrt(var)
    o = o - jnp.max(o, axis=-1, keepdims=True)
    out_ref[...] = o - jnp.log(jnp.sum(jnp.exp(o), axis=-1, keepdims=True))


def kernel(x, adj, W0m, b0m, W0v, b0v, W1m, b1m, W1v, b1v):
    sample = jax.random.normal(jax.random.key(42), (_N, _C), dtype=jnp.float32)
    e = adj.shape[1]
    epw = -(-e // (_NW * 4 * _CHUNK)) * 4 * _CHUNK
    ep = epw * _NW

    rowp = jnp.concatenate([adj[0], jnp.full((ep - e,), _NP - 1, jnp.int32)])
    colp = jnp.concatenate([adj[1], jnp.zeros((ep - e,), jnp.int32)])

    mesh = plsc.VectorSubcoreMesh(core_axis_name="c", subcore_axis_name="s")

    deg_call = pl.kernel(
        _deg_body,
        out_type=jax.ShapeDtypeStruct((_NC, _NS, 640), jnp.float32),
        mesh=mesh,
        scratch_types=(
            [pltpu.VMEM((_CHUNK,), jnp.int32)] * 4 +
            [pltpu.VMEM((_CHUNK,), jnp.float32),
             pltpu.VMEM((640,), jnp.float32),
             pltpu.VMEM_SHARED((_NP,), jnp.float32)] +
            [pltpu.SemaphoreType.DMA] * 8
        ),
    )
    pdeg = deg_call(rowp)
    pdeg = pdeg[:, :, :_ROWS_PT].reshape(_NC, _NP)
    deg2 = (pdeg[0, :_N] + pdeg[1, :_N] + 1.0).reshape(_N, 1)

    nb = 10
    bn = _N // nb
    mvs = pl.pallas_call(
        _dense_body,
        grid=(nb,),
        in_specs=[
            pl.BlockSpec((bn, _D), lambda i: (i, 0)),
            pl.BlockSpec((bn, 1), lambda i: (i, 0)),
            pl.BlockSpec((_D, _H), lambda i: (0, 0)),
            pl.BlockSpec((1, _H), lambda i: (0, 0)),
            pl.BlockSpec((_D, _H), lambda i: (0, 0)),
            pl.BlockSpec((1, _H), lambda i: (0, 0)),
            pl.BlockSpec((_H, _C), lambda i: (0, 0)),
            pl.BlockSpec((1, _C), lambda i: (0, 0)),
            pl.BlockSpec((_H, _C), lambda i: (0, 0)),
            pl.BlockSpec((1, _C), lambda i: (0, 0)),
        ],
        out_specs=pl.BlockSpec((bn, _P), lambda i: (i, 0)),
        out_shape=jax.ShapeDtypeStruct((_N, _P), jnp.float32),
    )(x, deg2, W0m, b0m.reshape(1, _H), W0v, b0v.reshape(1, _H),
      W1m, b1m.reshape(1, _C), W1v, b1v.reshape(1, _C))

    spmm_call = pl.kernel(
        _spmm_body,
        out_type=jax.ShapeDtypeStruct((_NC, _NP, _P), jnp.float32),
        mesh=mesh,
        scratch_types=(
            [pltpu.VMEM((_CHUNK,), jnp.int32)] * 8 +
            [pltpu.VMEM((_CHUNK, _P), jnp.float32)] * 5 +
            [pltpu.VMEM_SHARED((_NP, _P), jnp.float32)] +
            [pltpu.SemaphoreType.DMA] * 16
        ),
    )
    one = 1.0 + 0.0 * deg2[0, 0]
    pmv = spmm_call(rowp, colp, mvs, mvs * one)

    out = pl.pallas_call(
        _final_body,
        grid=(nb,),
        in_specs=[
            pl.BlockSpec((_NC, bn, _P), lambda i: (0, i, 0)),
            pl.BlockSpec((bn, _P), lambda i: (i, 0)),
            pl.BlockSpec((bn, 1), lambda i: (i, 0)),
            pl.BlockSpec((bn, _C), lambda i: (i, 0)),
        ],
        out_specs=pl.BlockSpec((bn, _C), lambda i: (i, 0)),
        out_shape=jax.ShapeDtypeStruct((_N, _C), jnp.float32),
    )(pmv, mvs, deg2, sample)
    return out


# speedup vs baseline: 1.4286x; 1.0313x over previous
import functools

import jax
import jax.numpy as jnp
from jax import lax
from jax.experimental import pallas as pl
from jax.experimental.pallas import tpu as pltpu
from jax.experimental.pallas import tpu_sc as plsc

_N = 10000
_D = 128
_H = 128
_C = 64
_P = 128

_NP = 10112
_NC = 2
_NS = 16
_NW = _NC * _NS
_CHUNK = 128
_ROWS_PT = _NP // _NS


def _elu(x):
    return jnp.where(x > 0, x, jnp.exp(x) - 1.0)


def _deg_body(row_hbm, out_hbm, idx0_v, idx1_v, idx2_v, idx3_v,
              ones_v, zrow_v, deg_sh,
              i0, i1, i2, i3, s0, s1, s2, s3):
    c = lax.axis_index("c")
    s = lax.axis_index("s")
    wid = c * _NS + s
    epw = row_hbm.shape[0] // _NW
    base = wid * epw
    idxs = (idx0_v, idx1_v, idx2_v, idx3_v)
    isems = (i0, i1, i2, i3)
    ssems = (s0, s1, s2, s3)

    def _z16(i, carry):
        zrow_v[pl.ds(i * 16, 16)] = jnp.zeros((16,), jnp.float32)
        return carry

    lax.fori_loop(0, 640 // 16, _z16, 0)
    for q in range(_CHUNK // 16):
        ones_v[pl.ds(q * 16, 16)] = jnp.ones((16,), jnp.float32)
    pltpu.sync_copy(zrow_v.at[pl.ds(0, _ROWS_PT)],
                    deg_sh.at[pl.ds(s * _ROWS_PT, _ROWS_PT)])
    plsc.subcore_barrier()

    def _group(g, carry):
        icps = [pltpu.async_copy(
            row_hbm.at[pl.ds(base + (g * 4 + b) * _CHUNK, _CHUNK)],
            idxs[b], isems[b]) for b in range(4)]
        scps = []
        for b in range(4):
            icps[b].wait()
            scps.append(pltpu.async_copy(
                ones_v, deg_sh.at[idxs[b]], ssems[b], add=True))
        for cp in scps:
            cp.wait()
        return carry

    lax.fori_loop(0, epw // _CHUNK // 4, _group, 0)
    plsc.subcore_barrier()

    pltpu.sync_copy(deg_sh.at[pl.ds(s * _ROWS_PT, _ROWS_PT)],
                    zrow_v.at[pl.ds(0, _ROWS_PT)])
    pltpu.sync_copy(zrow_v, out_hbm.at[c, s])


def _spmm_body(row_hbm, col_hbm, mvsa_hbm, mvsb_hbm, pmv_hbm,
               ridx0_v, ridx1_v, ridx2_v, ridx3_v,
               cidx0_v, cidx1_v, cidx2_v, cidx3_v,
               rows0_v, rows1_v, rows2_v, rows3_v, zbuf_v,
               acc_sh, i0, i1, i2, i3, j0, j1, j2, j3,
               g0, g1, g2, g3, s0, s1, s2, s3):
    c = lax.axis_index("c")
    s = lax.axis_index("s")
    wid = c * _NS + s
    epw = row_hbm.shape[0] // _NW
    base = wid * epw
    ridxs = (ridx0_v, ridx1_v, ridx2_v, ridx3_v)
    cidxs = (cidx0_v, cidx1_v, cidx2_v, cidx3_v)
    rows = (rows0_v, rows1_v, rows2_v, rows3_v)
    isems = (i0, i1, i2, i3)
    jsems = (j0, j1, j2, j3)
    gsems = (g0, g1, g2, g3)
    ssems = (s0, s1, s2, s3)

    def _z2d(i, carry):
        for q in range(_P // 16):
            zbuf_v[i, pl.ds(q * 16, 16)] = jnp.zeros((16,), jnp.float32)
        return carry

    lax.fori_loop(0, _CHUNK, _z2d, 0)
    for j in range(4):
        off = s * _ROWS_PT + j * _CHUNK
        pltpu.sync_copy(zbuf_v, acc_sh.at[pl.ds(off, _CHUNK)])
    off = s * _ROWS_PT + 4 * _CHUNK
    pltpu.sync_copy(zbuf_v.at[pl.ds(0, _ROWS_PT - 4 * _CHUNK)],
                    acc_sh.at[pl.ds(off, _ROWS_PT - 4 * _CHUNK)])
    plsc.subcore_barrier()

    tabs = (mvsa_hbm, mvsb_hbm)

    def _group(g, carry):
        ccps = [pltpu.async_copy(
            col_hbm.at[pl.ds(base + (g * 2 + b) * _CHUNK, _CHUNK)],
            cidxs[b], isems[b]) for b in range(2)]
        rcps = [pltpu.async_copy(
            row_hbm.at[pl.ds(base + (g * 2 + b) * _CHUNK, _CHUNK)],
            ridxs[b], jsems[b]) for b in range(2)]
        gcps = []
        for b in range(2):
            ccps[b].wait()
            gcps.append(pltpu.async_copy(
                tabs[b].at[cidxs[b]], rows[b], gsems[b]))
        scps = []
        for b in range(2):
            gcps[b].wait()
            rcps[b].wait()
            scps.append(pltpu.async_copy(
                rows[b], acc_sh.at[ridxs[b]], ssems[b], add=True))
        for cp in scps:
            cp.wait()
        return carry

    lax.fori_loop(0, epw // _CHUNK // 2, _group, 0)
    plsc.subcore_barrier()

    for j in range(4):
        off = s * _ROWS_PT + j * _CHUNK
        pltpu.sync_copy(acc_sh.at[pl.ds(off, _CHUNK)], zbuf_v)
        pltpu.sync_copy(zbuf_v, pmv_hbm.at[c, pl.ds(off, _CHUNK)])
    off = s * _ROWS_PT + 4 * _CHUNK
    tail = _ROWS_PT - 4 * _CHUNK
    pltpu.sync_copy(acc_sh.at[pl.ds(off, tail)], zbuf_v.at[pl.ds(0, tail)])
    pltpu.sync_copy(zbuf_v.at[pl.ds(0, tail)], pmv_hbm.at[c, pl.ds(off, tail)])


def _dense_body(x_ref, deg_ref, w0m_ref, b0m_ref, w0v_ref, b0v_ref,
                w1m_ref, b1m_ref, w1v_ref, b1v_ref, mvs_ref):
    dot = functools.partial(jnp.dot, preferred_element_type=jnp.float32,
                            precision=lax.Precision.HIGHEST)
    xb = x_ref[...]
    hm = _elu(dot(xb, w0m_ref[...]) + b0m_ref[...])
    hv = jnp.maximum(dot(xb, w0v_ref[...]) + b0v_ref[...], 0.0)
    m = _elu(dot(hm, w1m_ref[...]) + b1m_ref[...])
    v = jnp.maximum(dot(hv, w1v_ref[...]) + b1v_ref[...], 0.0) + 1e-6
    att = jnp.exp(-v)
    deg = deg_ref[...]
    dinv0 = lax.rsqrt(deg)
    dinv1 = 1.0 / deg
    mvs_ref[...] = jnp.concatenate(
        [dinv0 * (m * att), dinv1 * (v * att * att)], axis=1)


def _final_body(pmv_ref, mvs_ref, deg_ref, smp_ref, out_ref):
    deg = deg_ref[...]
    dinv0 = lax.rsqrt(deg)
    dinv1 = 1.0 / deg
    tot = pmv_ref[0] + pmv_ref[1] + mvs_ref[...]
    mean = dinv0 * tot[:, :_C]
    var = dinv1 * tot[:, _C:]
    o = mean + smp_ref[...] * jnp.sqrt(var)
    o = o - jnp.max(o, axis=-1, keepdims=True)
    out_ref[...] = o - jnp.log(jnp.sum(jnp.exp(o), axis=-1, keepdims=True))


def kernel(x, adj, W0m, b0m, W0v, b0v, W1m, b1m, W1v, b1v):
    sample = jax.random.normal(jax.random.key(42), (_N, _C), dtype=jnp.float32)
    e = adj.shape[1]
    epw = -(-e // (_NW * 4 * _CHUNK)) * 4 * _CHUNK
    ep = epw * _NW

    rowp = jnp.concatenate([adj[0], jnp.full((ep - e,), _NP - 1, jnp.int32)])
    colp = jnp.concatenate([adj[1], jnp.zeros((ep - e,), jnp.int32)])

    mesh = plsc.VectorSubcoreMesh(core_axis_name="c", subcore_axis_name="s")

    deg_call = pl.kernel(
        _deg_body,
        out_type=jax.ShapeDtypeStruct((_NC, _NS, 640), jnp.float32),
        mesh=mesh,
        scratch_types=(
            [pltpu.VMEM((_CHUNK,), jnp.int32)] * 4 +
            [pltpu.VMEM((_CHUNK,), jnp.float32),
             pltpu.VMEM((640,), jnp.float32),
             pltpu.VMEM_SHARED((_NP,), jnp.float32)] +
            [pltpu.SemaphoreType.DMA] * 8
        ),
    )
    pdeg = deg_call(rowp)
    pdeg = pdeg[:, :, :_ROWS_PT].reshape(_NC, _NP)
    deg2 = (pdeg[0, :_N] + pdeg[1, :_N] + 1.0).reshape(_N, 1)

    nb = 10
    bn = _N // nb
    mvs = pl.pallas_call(
        _dense_body,
        grid=(nb,),
        in_specs=[
            pl.BlockSpec((bn, _D), lambda i: (i, 0)),
            pl.BlockSpec((bn, 1), lambda i: (i, 0)),
            pl.BlockSpec((_D, _H), lambda i: (0, 0)),
            pl.BlockSpec((1, _H), lambda i: (0, 0)),
            pl.BlockSpec((_D, _H), lambda i: (0, 0)),
            pl.BlockSpec((1, _H), lambda i: (0, 0)),
            pl.BlockSpec((_H, _C), lambda i: (0, 0)),
            pl.BlockSpec((1, _C), lambda i: (0, 0)),
            pl.BlockSpec((_H, _C), lambda i: (0, 0)),
            pl.BlockSpec((1, _C), lambda i: (0, 0)),
        ],
        out_specs=pl.BlockSpec((bn, _P), lambda i: (i, 0)),
        out_shape=jax.ShapeDtypeStruct((_N, _P), jnp.float32),
    )(x, deg2, W0m, b0m.reshape(1, _H), W0v, b0v.reshape(1, _H),
      W1m, b1m.reshape(1, _C), W1v, b1v.reshape(1, _C))

    spmm_call = pl.kernel(
        _spmm_body,
        out_type=jax.ShapeDtypeStruct((_NC, _NP, _P), jnp.float32),
        mesh=mesh,
        scratch_types=(
            [pltpu.VMEM((_CHUNK,), jnp.int32)] * 8 +
            [pltpu.VMEM((_CHUNK, _P), jnp.float32)] * 5 +
            [pltpu.VMEM_SHARED((_NP, _P), jnp.float32)] +
            [pltpu.SemaphoreType.DMA] * 16
        ),
    )
    one = 1.0 + 0.0 * deg2[0, 0]
    pmv = spmm_call(rowp, colp, mvs, mvs * one)

    out = pl.pallas_call(
        _final_body,
        grid=(nb,),
        in_specs=[
            pl.BlockSpec((_NC, bn, _P), lambda i: (0, i, 0)),
            pl.BlockSpec((bn, _P), lambda i: (i, 0)),
            pl.BlockSpec((bn, 1), lambda i: (i, 0)),
            pl.BlockSpec((bn, _C), lambda i: (i, 0)),
        ],
        out_specs=pl.BlockSpec((bn, _C), lambda i: (i, 0)),
        out_shape=jax.ShapeDtypeStruct((_N, _C), jnp.float32),
    )(pmv, mvs, deg2, sample)
    return out


# speedup vs baseline: 1.5426x; 1.0798x over previous
import functools

import jax
import jax.numpy as jnp
from jax import lax
from jax.experimental import pallas as pl
from jax.experimental.pallas import tpu as pltpu
from jax.experimental.pallas import tpu_sc as plsc

_N = 10000
_D = 128
_H = 128
_C = 64
_P = 128

_NP = 10112
_NC = 2
_NS = 16
_NW = _NC * _NS
_CHUNK = 128
_ROWS_PT = _NP // _NS


def _elu(x):
    return jnp.where(x > 0, x, jnp.exp(x) - 1.0)


def _deg_body(row_hbm, out_hbm, idx0_v, idx1_v, idx2_v, idx3_v,
              ones_v, zrow_v, deg_sh,
              i0, i1, i2, i3, s0, s1, s2, s3):
    c = lax.axis_index("c")
    s = lax.axis_index("s")
    wid = c * _NS + s
    epw = row_hbm.shape[0] // _NW
    base = wid * epw
    idxs = (idx0_v, idx1_v, idx2_v, idx3_v)
    isems = (i0, i1, i2, i3)
    ssems = (s0, s1, s2, s3)

    def _z16(i, carry):
        zrow_v[pl.ds(i * 16, 16)] = jnp.zeros((16,), jnp.float32)
        return carry

    lax.fori_loop(0, 640 // 16, _z16, 0)
    for q in range(_CHUNK // 16):
        ones_v[pl.ds(q * 16, 16)] = jnp.ones((16,), jnp.float32)
    pltpu.sync_copy(zrow_v.at[pl.ds(0, _ROWS_PT)],
                    deg_sh.at[pl.ds(s * _ROWS_PT, _ROWS_PT)])
    plsc.subcore_barrier()

    def _group(g, carry):
        icps = [pltpu.async_copy(
            row_hbm.at[pl.ds(base + (g * 4 + b) * _CHUNK, _CHUNK)],
            idxs[b], isems[b]) for b in range(4)]
        scps = []
        for b in range(4):
            icps[b].wait()
            scps.append(pltpu.async_copy(
                ones_v, deg_sh.at[idxs[b]], ssems[b], add=True))
        for cp in scps:
            cp.wait()
        return carry

    lax.fori_loop(0, epw // _CHUNK // 4, _group, 0)
    plsc.subcore_barrier()

    pltpu.sync_copy(deg_sh.at[pl.ds(s * _ROWS_PT, _ROWS_PT)],
                    zrow_v.at[pl.ds(0, _ROWS_PT)])
    pltpu.sync_copy(zrow_v, out_hbm.at[c, s])


def _spmm_body(rc_hbm, mvsa_hbm, mvsb_hbm, pmv_hbm,
               pidx0_v, pidx1_v, cidx0_v, cidx1_v, sidx0_v, sidx1_v,
               rows0_v, rows1_v, zbuf_v,
               acc_sh, i0, i1, g0, g1, s0, s1):
    c = lax.axis_index("c")
    s = lax.axis_index("s")
    wid = c * _NS + s
    epw = rc_hbm.shape[0] // _NW
    base = wid * epw
    pidxs = (pidx0_v, pidx1_v)
    cidxs = (cidx0_v, cidx1_v)
    sidxs = (sidx0_v, sidx1_v)
    rows = (rows0_v, rows1_v)
    isems = (i0, i1)
    gsems = (g0, g1)
    ssems = (s0, s1)

    def _z2d(i, carry):
        for q in range(_P // 16):
            zbuf_v[i, pl.ds(q * 16, 16)] = jnp.zeros((16,), jnp.float32)
        return carry

    lax.fori_loop(0, _CHUNK, _z2d, 0)
    for j in range(4):
        off = s * _ROWS_PT + j * _CHUNK
        pltpu.sync_copy(zbuf_v, acc_sh.at[pl.ds(off, _CHUNK)])
    off = s * _ROWS_PT + 4 * _CHUNK
    pltpu.sync_copy(zbuf_v.at[pl.ds(0, _ROWS_PT - 4 * _CHUNK)],
                    acc_sh.at[pl.ds(off, _ROWS_PT - 4 * _CHUNK)])
    plsc.subcore_barrier()

    tabs = (mvsa_hbm, mvsb_hbm)

    def _group(g, carry):
        icps = [pltpu.async_copy(
            rc_hbm.at[pl.ds(base + (g * 2 + b) * _CHUNK, _CHUNK)],
            pidxs[b], isems[b]) for b in range(2)]
        gcps = []
        for b in range(2):
            icps[b].wait()
            for q in range(_CHUNK // 16):
                v = pidxs[b][pl.ds(q * 16, 16)]
                cidxs[b][pl.ds(q * 16, 16)] = jnp.bitwise_and(v, 16383)
                sidxs[b][pl.ds(q * 16, 16)] = lax.shift_right_logical(v, 14)
            gcps.append(pltpu.async_copy(
                tabs[b].at[cidxs[b]], rows[b], gsems[b]))
        scps = []
        for b in range(2):
            gcps[b].wait()
            scps.append(pltpu.async_copy(
                rows[b], acc_sh.at[sidxs[b]], ssems[b], add=True))
        for cp in scps:
            cp.wait()
        return carry

    lax.fori_loop(0, epw // _CHUNK // 2, _group, 0)
    plsc.subcore_barrier()

    for j in range(4):
        off = s * _ROWS_PT + j * _CHUNK
        pltpu.sync_copy(acc_sh.at[pl.ds(off, _CHUNK)], zbuf_v)
        pltpu.sync_copy(zbuf_v, pmv_hbm.at[c, pl.ds(off, _CHUNK)])
    off = s * _ROWS_PT + 4 * _CHUNK
    tail = _ROWS_PT - 4 * _CHUNK
    pltpu.sync_copy(acc_sh.at[pl.ds(off, tail)], zbuf_v.at[pl.ds(0, tail)])
    pltpu.sync_copy(zbuf_v.at[pl.ds(0, tail)], pmv_hbm.at[c, pl.ds(off, tail)])


def _dense_body(x_ref, deg_ref, w0m_ref, b0m_ref, w0v_ref, b0v_ref,
                w1m_ref, b1m_ref, w1v_ref, b1v_ref, mvs_ref):
    dot = functools.partial(jnp.dot, preferred_element_type=jnp.float32,
                            precision=lax.Precision.HIGHEST)
    xb = x_ref[...]
    hm = _elu(dot(xb, w0m_ref[...]) + b0m_ref[...])
    hv = jnp.maximum(dot(xb, w0v_ref[...]) + b0v_ref[...], 0.0)
    m = _elu(dot(hm, w1m_ref[...]) + b1m_ref[...])
    v = jnp.maximum(dot(hv, w1v_ref[...]) + b1v_ref[...], 0.0) + 1e-6
    att = jnp.exp(-v)
    deg = deg_ref[...]
    dinv0 = lax.rsqrt(deg)
    dinv1 = 1.0 / deg
    mvs_ref[...] = jnp.concatenate(
        [dinv0 * (m * att), dinv1 * (v * att * att)], axis=1)


def _final_body(pmv_ref, mvs_ref, deg_ref, smp_ref, out_ref):
    deg = deg_ref[...]
    dinv0 = lax.rsqrt(deg)
    dinv1 = 1.0 / deg
    tot = pmv_ref[0] + pmv_ref[1] + mvs_ref[...]
    mean = dinv0 * tot[:, :_C]
    var = dinv1 * tot[:, _C:]
    o = mean + smp_ref[...] * jnp.sqrt(var)
    o = o - jnp.max(o, axis=-1, keepdims=True)
    out_ref[...] = o - jnp.log(jnp.sum(jnp.exp(o), axis=-1, keepdims=True))


def kernel(x, adj, W0m, b0m, W0v, b0v, W1m, b1m, W1v, b1v):
    sample = jax.random.normal(jax.random.key(42), (_N, _C), dtype=jnp.float32)
    e = adj.shape[1]
    epw = -(-e // (_NW * 4 * _CHUNK)) * 4 * _CHUNK
    ep = epw * _NW

    rowp = jnp.concatenate([adj[0], jnp.full((ep - e,), _NP - 1, jnp.int32)])
    colp = jnp.concatenate([adj[1], jnp.zeros((ep - e,), jnp.int32)])

    mesh = plsc.VectorSubcoreMesh(core_axis_name="c", subcore_axis_name="s")

    deg_call = pl.kernel(
        _deg_body,
        out_type=jax.ShapeDtypeStruct((_NC, _NS, 640), jnp.float32),
        mesh=mesh,
        scratch_types=(
            [pltpu.VMEM((_CHUNK,), jnp.int32)] * 4 +
            [pltpu.VMEM((_CHUNK,), jnp.float32),
             pltpu.VMEM((640,), jnp.float32),
             pltpu.VMEM_SHARED((_NP,), jnp.float32)] +
            [pltpu.SemaphoreType.DMA] * 8
        ),
    )
    pdeg = deg_call(rowp)
    pdeg = pdeg[:, :, :_ROWS_PT].reshape(_NC, _NP)
    deg2 = (pdeg[0, :_N] + pdeg[1, :_N] + 1.0).reshape(_N, 1)

    nb = 10
    bn = _N // nb
    mvs = pl.pallas_call(
        _dense_body,
        grid=(nb,),
        in_specs=[
            pl.BlockSpec((bn, _D), lambda i: (i, 0)),
            pl.BlockSpec((bn, 1), lambda i: (i, 0)),
            pl.BlockSpec((_D, _H), lambda i: (0, 0)),
            pl.BlockSpec((1, _H), lambda i: (0, 0)),
            pl.BlockSpec((_D, _H), lambda i: (0, 0)),
            pl.BlockSpec((1, _H), lambda i: (0, 0)),
            pl.BlockSpec((_H, _C), lambda i: (0, 0)),
            pl.BlockSpec((1, _C), lambda i: (0, 0)),
            pl.BlockSpec((_H, _C), lambda i: (0, 0)),
            pl.BlockSpec((1, _C), lambda i: (0, 0)),
        ],
        out_specs=pl.BlockSpec((bn, _P), lambda i: (i, 0)),
        out_shape=jax.ShapeDtypeStruct((_N, _P), jnp.float32),
    )(x, deg2, W0m, b0m.reshape(1, _H), W0v, b0v.reshape(1, _H),
      W1m, b1m.reshape(1, _C), W1v, b1v.reshape(1, _C))

    spmm_call = pl.kernel(
        _spmm_body,
        out_type=jax.ShapeDtypeStruct((_NC, _NP, _P), jnp.float32),
        mesh=mesh,
        scratch_types=(
            [pltpu.VMEM((_CHUNK,), jnp.int32)] * 6 +
            [pltpu.VMEM((_CHUNK, _P), jnp.float32)] * 3 +
            [pltpu.VMEM_SHARED((_NP, _P), jnp.float32)] +
            [pltpu.SemaphoreType.DMA] * 6
        ),
    )
    one = 1.0 + 0.0 * deg2[0, 0]
    rc = rowp * 16384 + colp
    pmv = spmm_call(rc, mvs, mvs * one)

    out = pl.pallas_call(
        _final_body,
        grid=(nb,),
        in_specs=[
            pl.BlockSpec((_NC, bn, _P), lambda i: (0, i, 0)),
            pl.BlockSpec((bn, _P), lambda i: (i, 0)),
            pl.BlockSpec((bn, 1), lambda i: (i, 0)),
            pl.BlockSpec((bn, _C), lambda i: (i, 0)),
        ],
        out_specs=pl.BlockSpec((bn, _C), lambda i: (i, 0)),
        out_shape=jax.ShapeDtypeStruct((_N, _C), jnp.float32),
    )(pmv, mvs, deg2, sample)
    return out


# speedup vs baseline: 1.6325x; 1.0582x over previous
import functools

import jax
import jax.numpy as jnp
from jax import lax
from jax.experimental import pallas as pl
from jax.experimental.pallas import tpu as pltpu
from jax.experimental.pallas import tpu_sc as plsc

_N = 10000
_D = 128
_H = 128
_C = 64
_P = 128

_NP = 10112
_NC = 2
_NS = 16
_NW = _NC * _NS
_CHUNK = 128
_ROWS_PT = _NP // _NS


def _elu(x):
    return jnp.where(x > 0, x, jnp.exp(x) - 1.0)


def _deg_body(row_hbm, out_hbm, idx0_v, idx1_v, idx2_v, idx3_v,
              ones_v, zrow_v, deg_sh,
              i0, i1, i2, i3, s0, s1, s2, s3):
    c = lax.axis_index("c")
    s = lax.axis_index("s")
    wid = c * _NS + s
    epw = row_hbm.shape[0] // _NW
    base = wid * epw
    idxs = (idx0_v, idx1_v, idx2_v, idx3_v)
    isems = (i0, i1, i2, i3)
    ssems = (s0, s1, s2, s3)

    def _z16(i, carry):
        zrow_v[pl.ds(i * 16, 16)] = jnp.zeros((16,), jnp.float32)
        return carry

    lax.fori_loop(0, 640 // 16, _z16, 0)
    for q in range(_CHUNK // 16):
        ones_v[pl.ds(q * 16, 16)] = jnp.ones((16,), jnp.float32)
    pltpu.sync_copy(zrow_v.at[pl.ds(0, _ROWS_PT)],
                    deg_sh.at[pl.ds(s * _ROWS_PT, _ROWS_PT)])
    plsc.subcore_barrier()

    def _group(g, carry):
        icps = [pltpu.async_copy(
            row_hbm.at[pl.ds(base + (g * 4 + b) * _CHUNK, _CHUNK)],
            idxs[b], isems[b]) for b in range(4)]
        scps = []
        for b in range(4):
            icps[b].wait()
            scps.append(pltpu.async_copy(
                ones_v, deg_sh.at[idxs[b]], ssems[b], add=True))
        for cp in scps:
            cp.wait()
        return carry

    lax.fori_loop(0, epw // _CHUNK // 4, _group, 0)
    plsc.subcore_barrier()

    pltpu.sync_copy(deg_sh.at[pl.ds(s * _ROWS_PT, _ROWS_PT)],
                    zrow_v.at[pl.ds(0, _ROWS_PT)])
    pltpu.sync_copy(zrow_v, out_hbm.at[c, s])


def _spmm_body(rc_hbm, mvsa_hbm, mvsb_hbm, pmv_hbm,
               pidx0_v, pidx1_v, cidx0_v, cidx1_v, sidx0_v, sidx1_v,
               rows0_v, rows1_v, zbuf_v,
               acc_sh, i0, i1, g0, g1, s0, s1):
    c = lax.axis_index("c")
    s = lax.axis_index("s")
    wid = c * _NS + s
    epw = rc_hbm.shape[0] // _NW
    base = wid * epw
    pidxs = (pidx0_v, pidx1_v)
    cidxs = (cidx0_v, cidx1_v)
    sidxs = (sidx0_v, sidx1_v)
    rows = (rows0_v, rows1_v)
    isems = (i0, i1)
    gsems = (g0, g1)
    ssems = (s0, s1)

    def _z2d(i, carry):
        for q in range(_P // 16):
            zbuf_v[i, pl.ds(q * 16, 16)] = jnp.zeros((16,), jnp.float32)
        return carry

    lax.fori_loop(0, _CHUNK, _z2d, 0)
    for j in range(4):
        off = s * _ROWS_PT + j * _CHUNK
        pltpu.sync_copy(zbuf_v, acc_sh.at[pl.ds(off, _CHUNK)])
    off = s * _ROWS_PT + 4 * _CHUNK
    pltpu.sync_copy(zbuf_v.at[pl.ds(0, _ROWS_PT - 4 * _CHUNK)],
                    acc_sh.at[pl.ds(off, _ROWS_PT - 4 * _CHUNK)])
    plsc.subcore_barrier()

    tabs = (mvsa_hbm, mvsb_hbm)

    def _group(g, carry):
        icps = [pltpu.async_copy(
            rc_hbm.at[pl.ds(base + (g * 2 + b) * _CHUNK, _CHUNK)],
            pidxs[b], isems[b]) for b in range(2)]
        gcps = []
        for b in range(2):
            icps[b].wait()
            for q in range(_CHUNK // 16):
                v = pidxs[b][pl.ds(q * 16, 16)]
                cidxs[b][pl.ds(q * 16, 16)] = jnp.bitwise_and(v, 16383)
                sidxs[b][pl.ds(q * 16, 16)] = lax.shift_right_logical(v, 14)
            gcps.append(pltpu.async_copy(
                tabs[b].at[cidxs[b]], rows[b], gsems[b]))
        scps = []
        for b in range(2):
            gcps[b].wait()
            scps.append(pltpu.async_copy(
                rows[b], acc_sh.at[sidxs[b]], ssems[b], add=True))
        for cp in scps:
            cp.wait()
        return carry

    lax.fori_loop(0, epw // _CHUNK // 2, _group, 0)
    plsc.subcore_barrier()

    for j in range(4):
        off = s * _ROWS_PT + j * _CHUNK
        pltpu.sync_copy(acc_sh.at[pl.ds(off, _CHUNK)], zbuf_v)
        pltpu.sync_copy(zbuf_v, pmv_hbm.at[c, pl.ds(off, _CHUNK)])
    off = s * _ROWS_PT + 4 * _CHUNK
    tail = _ROWS_PT - 4 * _CHUNK
    pltpu.sync_copy(acc_sh.at[pl.ds(off, tail)], zbuf_v.at[pl.ds(0, tail)])
    pltpu.sync_copy(zbuf_v.at[pl.ds(0, tail)], pmv_hbm.at[c, pl.ds(off, tail)])


def _dense_body(x_ref, deg_ref, w0m_ref, b0m_ref, w0v_ref, b0v_ref,
                w1m_ref, b1m_ref, w1v_ref, b1v_ref, mvs_ref):
    dot = functools.partial(jnp.dot, preferred_element_type=jnp.float32)
    xb = x_ref[...]
    hm = _elu(dot(xb, w0m_ref[...]) + b0m_ref[...])
    hv = jnp.maximum(dot(xb, w0v_ref[...]) + b0v_ref[...], 0.0)
    m = _elu(dot(hm, w1m_ref[...]) + b1m_ref[...])
    v = jnp.maximum(dot(hv, w1v_ref[...]) + b1v_ref[...], 0.0) + 1e-6
    att = jnp.exp(-v)
    deg = deg_ref[...]
    dinv0 = lax.rsqrt(deg)
    dinv1 = 1.0 / deg
    mvs_ref[...] = jnp.concatenate(
        [dinv0 * (m * att), dinv1 * (v * att * att)], axis=1)


def _final_body(pmv_ref, mvs_ref, deg_ref, smp_ref, out_ref):
    deg = deg_ref[...]
    dinv0 = lax.rsqrt(deg)
    dinv1 = 1.0 / deg
    tot = pmv_ref[0] + pmv_ref[1] + mvs_ref[...]
    mean = dinv0 * tot[:, :_C]
    var = dinv1 * tot[:, _C:]
    o = mean + smp_ref[...] * jnp.sqrt(var)
    o = o - jnp.max(o, axis=-1, keepdims=True)
    out_ref[...] = o - jnp.log(jnp.sum(jnp.exp(o), axis=-1, keepdims=True))


def kernel(x, adj, W0m, b0m, W0v, b0v, W1m, b1m, W1v, b1v):
    sample = jax.random.normal(jax.random.key(42), (_N, _C), dtype=jnp.float32)
    e = adj.shape[1]
    epw = -(-e // (_NW * 4 * _CHUNK)) * 4 * _CHUNK
    ep = epw * _NW

    rowp = jnp.concatenate([adj[0], jnp.full((ep - e,), _NP - 1, jnp.int32)])
    colp = jnp.concatenate([adj[1], jnp.zeros((ep - e,), jnp.int32)])

    mesh = plsc.VectorSubcoreMesh(core_axis_name="c", subcore_axis_name="s")

    deg_call = pl.kernel(
        _deg_body,
        out_type=jax.ShapeDtypeStruct((_NC, _NS, 640), jnp.float32),
        mesh=mesh,
        scratch_types=(
            [pltpu.VMEM((_CHUNK,), jnp.int32)] * 4 +
            [pltpu.VMEM((_CHUNK,), jnp.float32),
             pltpu.VMEM((640,), jnp.float32),
             pltpu.VMEM_SHARED((_NP,), jnp.float32)] +
            [pltpu.SemaphoreType.DMA] * 8
        ),
    )
    pdeg = deg_call(rowp)
    pdeg = pdeg[:, :, :_ROWS_PT].reshape(_NC, _NP)
    deg2 = (pdeg[0, :_N] + pdeg[1, :_N] + 1.0).reshape(_N, 1)

    nb = 10
    bn = _N // nb
    mvs = pl.pallas_call(
        _dense_body,
        grid=(nb,),
        in_specs=[
            pl.BlockSpec((bn, _D), lambda i: (i, 0)),
            pl.BlockSpec((bn, 1), lambda i: (i, 0)),
            pl.BlockSpec((_D, _H), lambda i: (0, 0)),
            pl.BlockSpec((1, _H), lambda i: (0, 0)),
            pl.BlockSpec((_D, _H), lambda i: (0, 0)),
            pl.BlockSpec((1, _H), lambda i: (0, 0)),
            pl.BlockSpec((_H, _C), lambda i: (0, 0)),
            pl.BlockSpec((1, _C), lambda i: (0, 0)),
            pl.BlockSpec((_H, _C), lambda i: (0, 0)),
            pl.BlockSpec((1, _C), lambda i: (0, 0)),
        ],
        out_specs=pl.BlockSpec((bn, _P), lambda i: (i, 0)),
        out_shape=jax.ShapeDtypeStruct((_N, _P), jnp.float32),
    )(x, deg2, W0m, b0m.reshape(1, _H), W0v, b0v.reshape(1, _H),
      W1m, b1m.reshape(1, _C), W1v, b1v.reshape(1, _C))

    spmm_call = pl.kernel(
        _spmm_body,
        out_type=jax.ShapeDtypeStruct((_NC, _NP, _P), jnp.float32),
        mesh=mesh,
        scratch_types=(
            [pltpu.VMEM((_CHUNK,), jnp.int32)] * 6 +
            [pltpu.VMEM((_CHUNK, _P), jnp.float32)] * 3 +
            [pltpu.VMEM_SHARED((_NP, _P), jnp.float32)] +
            [pltpu.SemaphoreType.DMA] * 6
        ),
    )
    one = 1.0 + 0.0 * deg2[0, 0]
    rc = rowp * 16384 + colp
    pmv = spmm_call(rc, mvs, mvs * one)

    out = pl.pallas_call(
        _final_body,
        grid=(nb,),
        in_specs=[
            pl.BlockSpec((_NC, bn, _P), lambda i: (0, i, 0)),
            pl.BlockSpec((bn, _P), lambda i: (i, 0)),
            pl.BlockSpec((bn, 1), lambda i: (i, 0)),
            pl.BlockSpec((bn, _C), lambda i: (i, 0)),
        ],
        out_specs=pl.BlockSpec((bn, _C), lambda i: (i, 0)),
        out_shape=jax.ShapeDtypeStruct((_N, _C), jnp.float32),
    )(pmv, mvs, deg2, sample)
    return out
